# baseline (speedup 1.0000x reference)
import functools

import jax
import jax.numpy as jnp
from jax.experimental import pallas as pl
from jax.experimental.pallas import tpu as pltpu

_K1 = 0.01
_K2 = 0.03
_DATA_RANGE = 1.0
_BETAS = (0.0448, 0.2856, 0.3001, 0.2363, 0.1333)
_N_SCALES = len(_BETAS)
_KW = 11

_NB = 4


def _wo_pad(wo):
    return 256 if wo > 128 else 128


def _band_right(w, wo, wp):
    j = jax.lax.broadcasted_iota(jnp.int32, (w, wp), 0)
    o = jax.lax.broadcasted_iota(jnp.int32, (w, wp), 1)
    return ((j >= o) & (j < o + _KW) & (o < wo)).astype(jnp.float32)


def _band_left(ho, h):
    o = jax.lax.broadcasted_iota(jnp.int32, (ho, h), 0)
    j = jax.lax.broadcasted_iota(jnp.int32, (ho, h), 1)
    return ((j >= o) & (j < o + _KW)).astype(jnp.float32)


def _pool_cols(w):
    j = jax.lax.broadcasted_iota(jnp.int32, (w, w // 2), 0)
    o = jax.lax.broadcasted_iota(jnp.int32, (w, w // 2), 1)
    return jnp.where((j >= 2 * o) & (j < 2 * o + 2), 0.5, 0.0).astype(jnp.float32)


def _pool_rows(h):
    i = jax.lax.broadcasted_iota(jnp.int32, (h // 2, h), 0)
    j = jax.lax.broadcasted_iota(jnp.int32, (h // 2, h), 1)
    return jnp.where((j >= 2 * i) & (j < 2 * i + 2), 0.5, 0.0).astype(jnp.float32)


def _loss_kernel(x_ref, y_ref, out_ref, *scratch, nb):
    c1 = (_K1 * _DATA_RANGE) ** 2
    c2 = (_K2 * _DATA_RANGE) ** 2
    inv_n = 1.0 / float(3 * _KW * _KW)
    h0 = x_ref.shape[0] // (nb * 3)

    refs = [(x_ref, y_ref)] + [
        (scratch[2 * i], scratch[2 * i + 1]) for i in range(_N_SCALES - 1)
    ]
    for s in range(_N_SCALES):
        xr, yr = refs[s]
        if s == 0:
            x = xr[...].reshape(nb, 3, h0, x_ref.shape[1])
            y = yr[...].reshape(nb, 3, h0, x_ref.shape[1])
        else:
            x = xr[...]
            y = yr[...]
        if s == 0:
            l1 = jnp.sum(jnp.abs(x - y))
            out_ref[0, _N_SCALES * nb:_N_SCALES * nb + 1, 0:1] = (
                l1.reshape(1, 1))
        hs, ws = x.shape[-2], x.shape[-1]
        ho, wo = hs - _KW + 1, ws - _KW + 1
        wp = _wo_pad(wo)

        x0, x1, x2 = x[:, 0], x[:, 1], x[:, 2]
        y0, y1, y2 = y[:, 0], y[:, 1], y[:, 2]
        zx = x0 + x1 + x2
        zy = y0 + y1 + y2
        zxx = x0 * x0 + x1 * x1 + x2 * x2
        zyy = y0 * y0 + y1 * y1 + y2 * y2
        zxy = x0 * y0 + x1 * y1 + x2 * y2

        aw = _band_right(ws, wo, wp)
        ah = _band_left(ho, hs)

        z2 = jnp.concatenate(
            [m.reshape(nb * hs, ws) for m in (zx, zy, zxx, zyy, zxy)], axis=0
        )
        zw = jnp.dot(z2, aw, preferred_element_type=jnp.float32)

        for n in range(nb):
            wins = [
                jnp.dot(ah, zw[(m * nb + n) * hs:(m * nb + n + 1) * hs, :],
                        preferred_element_type=jnp.float32) * inv_n
                for m in range(5)
            ]
            mu_x, mu_y, exx, eyy, exy = wins
            sxx = exx - mu_x * mu_x
            syy = eyy - mu_y * mu_y
            sxy = exy - mu_x * mu_y
            cs_map = (2.0 * sxy + c2) / (sxx + syy + c2)
            if s == _N_SCALES - 1:
                cs_map = (2.0 * mu_x * mu_y + c1) / (
                    mu_x * mu_x + mu_y * mu_y + c1) * cs_map
            rv = jnp.sum(cs_map, axis=0, keepdims=True)
            if wp == 256:
                rv = rv[:, :128] + rv[:, 128:]
            out_ref[0, s * nb + n:s * nb + n + 1, :] = rv

        if s < _N_SCALES - 1:
            h2 = hs // 2
            pw = _pool_cols(ws)
            ph = _pool_rows(hs)
            cat = jnp.concatenate(
                [x.reshape(nb * 3 * hs, ws), y.reshape(nb * 3 * hs, ws)],
                axis=0)
            t = jnp.dot(cat, pw, preferred_element_type=jnp.float32)
            for arr in range(2):
                dst = refs[s + 1][arr]
                for n in range(nb):
                    for c in range(3):
                        p = (arr * nb + n) * 3 + c
                        dst[n, c] = jnp.dot(
                            ph, t[p * hs:(p + 1) * hs, :],
                            preferred_element_type=jnp.float32)


def kernel(preds, target):
    preds = preds.astype(jnp.float32)
    target = target.astype(jnp.float32)
    B, C, D, H, W = preds.shape
    bc = B * C
    nb = _NB
    nblk = bc // nb

    x = preds.reshape(bc * D * H, W)
    y = target.reshape(bc * D * H, W)
    rows = nb * D * H
    out = pl.pallas_call(
        functools.partial(_loss_kernel, nb=nb),
        out_shape=jax.ShapeDtypeStruct((nblk, 32, 128), jnp.float32),
        grid=(nblk,),
        in_specs=[
            pl.BlockSpec((rows, W), lambda i: (i, 0)),
            pl.BlockSpec((rows, W), lambda i: (i, 0)),
        ],
        out_specs=pl.BlockSpec((1, 32, 128), lambda i: (i, 0, 0)),
        scratch_shapes=[
            pltpu.VMEM((nb, D, (H // 2) >> i, (W // 2) >> i), jnp.float32)
            for i in range(_N_SCALES - 1) for _ in range(2)
        ],
        compiler_params=pltpu.CompilerParams(
            dimension_semantics=("parallel",),
            allow_input_fusion=[True, True]),
    )(x, y)

    abs_sum = jnp.sum(out[:, _N_SCALES * nb, 0])
    mcs_cols = []
    for s in range(_N_SCALES):
        hs = H >> s
        ho, wo = hs - _KW + 1, hs - _KW + 1
        wp = _wo_pad(wo)
        rows = out[:, s * nb:(s + 1) * nb, :]
        tot = jnp.sum(rows, axis=-1).reshape(bc)
        tot = tot - float(ho * (wp - wo))
        mcs_cols.append(tot * (1.0 / float(ho * wo)))
    mcs = jnp.stack(mcs_cols, axis=-1).reshape(B, C, _N_SCALES).mean(axis=1)
    mcs = jnp.maximum(mcs, 1e-6)
    betas = jnp.asarray(_BETAS, jnp.float32)[None, :]
    ms = jnp.mean(jnp.prod(mcs ** betas, axis=1))
    n = float(preds.size)
    loss = jnp.float32(0.0)
    loss = loss + 0.5 * (abs_sum / n)
    loss = loss + 0.5 * (1.0 - ms)
    return loss


# speedup vs baseline: 1.0629x; 1.0629x over previous
import functools

import jax
import jax.numpy as jnp
from jax.experimental import pallas as pl
from jax.experimental.pallas import tpu as pltpu

_K1 = 0.01
_K2 = 0.03
_DATA_RANGE = 1.0
_BETAS = (0.0448, 0.2856, 0.3001, 0.2363, 0.1333)
_N_SCALES = len(_BETAS)
_KW = 11

_NB = 8


def _wo_pad(wo):
    return 256 if wo > 128 else 128


def _band_right(w, wo, wp):
    j = jax.lax.broadcasted_iota(jnp.int32, (w, wp), 0)
    o = jax.lax.broadcasted_iota(jnp.int32, (w, wp), 1)
    return ((j >= o) & (j < o + _KW) & (o < wo)).astype(jnp.float32)


def _band_left(ho, h):
    o = jax.lax.broadcasted_iota(jnp.int32, (ho, h), 0)
    j = jax.lax.broadcasted_iota(jnp.int32, (ho, h), 1)
    return ((j >= o) & (j < o + _KW)).astype(jnp.float32)


def _pool_cols(w):
    j = jax.lax.broadcasted_iota(jnp.int32, (w, w // 2), 0)
    o = jax.lax.broadcasted_iota(jnp.int32, (w, w // 2), 1)
    return jnp.where((j >= 2 * o) & (j < 2 * o + 2), 0.5, 0.0).astype(jnp.float32)


def _pool_rows(h):
    i = jax.lax.broadcasted_iota(jnp.int32, (h // 2, h), 0)
    j = jax.lax.broadcasted_iota(jnp.int32, (h // 2, h), 1)
    return jnp.where((j >= 2 * i) & (j < 2 * i + 2), 0.5, 0.0).astype(jnp.float32)


def _loss_kernel(x_ref, y_ref, out_ref, *scratch, nb):
    c1 = (_K1 * _DATA_RANGE) ** 2
    c2 = (_K2 * _DATA_RANGE) ** 2
    inv_n = 1.0 / float(3 * _KW * _KW)
    h0 = x_ref.shape[0] // (nb * 3)

    refs = [(x_ref, y_ref)] + [
        (scratch[2 * i], scratch[2 * i + 1]) for i in range(_N_SCALES - 1)
    ]
    for s in range(_N_SCALES):
        xr, yr = refs[s]
        if s == 0:
            x = xr[...].reshape(nb, 3, h0, x_ref.shape[1])
            y = yr[...].reshape(nb, 3, h0, x_ref.shape[1])
        else:
            x = xr[...]
            y = yr[...]
        if s == 0:
            l1 = jnp.sum(jnp.abs(x - y))
            out_ref[0, _N_SCALES * nb:_N_SCALES * nb + 1, 0:1] = (
                l1.reshape(1, 1))
        hs, ws = x.shape[-2], x.shape[-1]
        ho, wo = hs - _KW + 1, ws - _KW + 1
        wp = _wo_pad(wo)

        x0, x1, x2 = x[:, 0], x[:, 1], x[:, 2]
        y0, y1, y2 = y[:, 0], y[:, 1], y[:, 2]
        zx = x0 + x1 + x2
        zy = y0 + y1 + y2
        zxx = x0 * x0 + x1 * x1 + x2 * x2
        zyy = y0 * y0 + y1 * y1 + y2 * y2
        zxy = x0 * y0 + x1 * y1 + x2 * y2

        aw = _band_right(ws, wo, wp)
        ah = _band_left(ho, hs)

        z2 = jnp.concatenate(
            [m.reshape(nb * hs, ws) for m in (zx, zy, zxx, zyy, zxy)], axis=0
        )
        zw = jnp.dot(z2, aw, preferred_element_type=jnp.float32)

        for n in range(nb):
            wins = [
                jnp.dot(ah, zw[(m * nb + n) * hs:(m * nb + n + 1) * hs, :],
                        preferred_element_type=jnp.float32) * inv_n
                for m in range(5)
            ]
            mu_x, mu_y, exx, eyy, exy = wins
            sxx = exx - mu_x * mu_x
            syy = eyy - mu_y * mu_y
            sxy = exy - mu_x * mu_y
            cs_map = (2.0 * sxy + c2) / (sxx + syy + c2)
            if s == _N_SCALES - 1:
                cs_map = (2.0 * mu_x * mu_y + c1) / (
                    mu_x * mu_x + mu_y * mu_y + c1) * cs_map
            rv = jnp.sum(cs_map, axis=0, keepdims=True)
            if wp == 256:
                rv = rv[:, :128] + rv[:, 128:]
            out_ref[0, s * nb + n:s * nb + n + 1, :] = rv

        if s < _N_SCALES - 1:
            h2 = hs // 2
            pw = _pool_cols(ws)
            ph = _pool_rows(hs)
            cat = jnp.concatenate(
                [x.reshape(nb * 3 * hs, ws), y.reshape(nb * 3 * hs, ws)],
                axis=0)
            t = jnp.dot(cat, pw, preferred_element_type=jnp.float32)
            for arr in range(2):
                dst = refs[s + 1][arr]
                for n in range(nb):
                    for c in range(3):
                        p = (arr * nb + n) * 3 + c
                        dst[n, c] = jnp.dot(
                            ph, t[p * hs:(p + 1) * hs, :],
                            preferred_element_type=jnp.float32)


def kernel(preds, target):
    preds = preds.astype(jnp.float32)
    target = target.astype(jnp.float32)
    B, C, D, H, W = preds.shape
    bc = B * C
    nb = _NB
    nblk = bc // nb

    x = preds.reshape(bc * D * H, W)
    y = target.reshape(bc * D * H, W)
    rows = nb * D * H
    out = pl.pallas_call(
        functools.partial(_loss_kernel, nb=nb),
        out_shape=jax.ShapeDtypeStruct((nblk, 48, 128), jnp.float32),
        grid=(nblk,),
        in_specs=[
            pl.BlockSpec((rows, W), lambda i: (i, 0)),
            pl.BlockSpec((rows, W), lambda i: (i, 0)),
        ],
        out_specs=pl.BlockSpec((1, 48, 128), lambda i: (i, 0, 0)),
        scratch_shapes=[
            pltpu.VMEM((nb, D, (H // 2) >> i, (W // 2) >> i), jnp.float32)
            for i in range(_N_SCALES - 1) for _ in range(2)
        ],
        compiler_params=pltpu.CompilerParams(
            dimension_semantics=("parallel",),
            allow_input_fusion=[True, True]),
    )(x, y)

    abs_sum = jnp.sum(out[:, _N_SCALES * nb, 0])
    mcs_cols = []
    for s in range(_N_SCALES):
        hs = H >> s
        ho, wo = hs - _KW + 1, hs - _KW + 1
        wp = _wo_pad(wo)
        rows = out[:, s * nb:(s + 1) * nb, :]
        tot = jnp.sum(rows, axis=-1).reshape(bc)
        tot = tot - float(ho * (wp - wo))
        mcs_cols.append(tot * (1.0 / float(ho * wo)))
    mcs = jnp.stack(mcs_cols, axis=-1).reshape(B, C, _N_SCALES).mean(axis=1)
    mcs = jnp.maximum(mcs, 1e-6)
    betas = jnp.asarray(_BETAS, jnp.float32)[None, :]
    ms = jnp.mean(jnp.prod(mcs ** betas, axis=1))
    n = float(preds.size)
    loss = jnp.float32(0.0)
    loss = loss + 0.5 * (abs_sum / n)
    loss = loss + 0.5 * (1.0 - ms)
    return loss


# speedup vs baseline: 1.0737x; 1.0102x over previous
import functools

import jax
import jax.numpy as jnp
from jax.experimental import pallas as pl
from jax.experimental.pallas import tpu as pltpu

_K1 = 0.01
_K2 = 0.03
_DATA_RANGE = 1.0
_BETAS = (0.0448, 0.2856, 0.3001, 0.2363, 0.1333)
_N_SCALES = len(_BETAS)
_KW = 11

_NB = 8


def _wo_pad(wo):
    return 256 if wo > 128 else 128


def _band_right(w, wo, wp):
    j = jax.lax.broadcasted_iota(jnp.int32, (w, wp), 0)
    o = jax.lax.broadcasted_iota(jnp.int32, (w, wp), 1)
    return ((j >= o) & (j < o + _KW) & (o < wo)).astype(jnp.float32)


def _band_left(ho, h):
    o = jax.lax.broadcasted_iota(jnp.int32, (ho, h), 0)
    j = jax.lax.broadcasted_iota(jnp.int32, (ho, h), 1)
    return ((j >= o) & (j < o + _KW)).astype(jnp.float32)


def _pool_cols(w_store, w):
    j = jax.lax.broadcasted_iota(jnp.int32, (w_store, 128), 0)
    o = jax.lax.broadcasted_iota(jnp.int32, (w_store, 128), 1)
    sel = (j >= 2 * o) & (j < 2 * o + 2) & (o < w // 2) & (j < w)
    return jnp.where(sel, 0.25, 0.0).astype(jnp.float32)


def _loss_kernel(x_ref, y_ref, out_ref, *scratch, nb, hw):
    c1 = (_K1 * _DATA_RANGE) ** 2
    c2 = (_K2 * _DATA_RANGE) ** 2
    inv_n = 1.0 / float(3 * _KW * _KW)

    refs = [(x_ref, y_ref)] + [
        (scratch[2 * i], scratch[2 * i + 1]) for i in range(_N_SCALES - 1)
    ]
    tbufs = scratch[2 * (_N_SCALES - 1):]
    for s in range(_N_SCALES):
        hs = hw >> s
        ws = hw >> s
        stw = ws if s == 0 else 128
        xr, yr = refs[s]
        x = xr[...].reshape(nb, 3, hs, stw)
        y = yr[...].reshape(nb, 3, hs, stw)
        if s == 0:
            l1 = jnp.sum(jnp.abs(x - y))
            out_ref[0, _N_SCALES * nb:_N_SCALES * nb + 1, 0:1] = (
                l1.reshape(1, 1))
        ho, wo = hs - _KW + 1, ws - _KW + 1
        wp = _wo_pad(wo)

        x0, x1, x2 = x[:, 0], x[:, 1], x[:, 2]
        y0, y1, y2 = y[:, 0], y[:, 1], y[:, 2]
        zx = x0 + x1 + x2
        zy = y0 + y1 + y2
        zxx = x0 * x0 + x1 * x1 + x2 * x2
        zyy = y0 * y0 + y1 * y1 + y2 * y2
        zxy = x0 * y0 + x1 * y1 + x2 * y2

        aw = _band_right(stw, wo, wp)
        ah = _band_left(ho, hs)

        z2 = jnp.concatenate(
            [m.reshape(nb * hs, stw) for m in (zx, zy, zxx, zyy, zxy)], axis=0
        )
        zw = jnp.dot(z2, aw, preferred_element_type=jnp.float32)

        for n in range(nb):
            wins = [
                jnp.dot(ah, zw[(m * nb + n) * hs:(m * nb + n + 1) * hs, :],
                        preferred_element_type=jnp.float32) * inv_n
                for m in range(5)
            ]
            mu_x, mu_y, exx, eyy, exy = wins
            sxx = exx - mu_x * mu_x
            syy = eyy - mu_y * mu_y
            sxy = exy - mu_x * mu_y
            cs_map = (2.0 * sxy + c2) / (sxx + syy + c2)
            if s == _N_SCALES - 1:
                cs_map = (2.0 * mu_x * mu_y + c1) / (
                    mu_x * mu_x + mu_y * mu_y + c1) * cs_map
            rv = jnp.sum(cs_map, axis=0, keepdims=True)
            if wp == 256:
                rv = rv[:, :128] + rv[:, 128:]
            out_ref[0, s * nb + n:s * nb + n + 1, :] = rv

        if s < _N_SCALES - 1:
            h2 = hs // 2
            pw = _pool_cols(stw, ws)
            cat = jnp.concatenate(
                [x.reshape(nb * 3 * hs, stw), y.reshape(nb * 3 * hs, stw)],
                axis=0)
            tb = tbufs[s]
            tb[...] = jnp.dot(cat, pw, preferred_element_type=jnp.float32)
            pooled = tb[0::2, :] + tb[1::2, :]
            half = nb * 3 * h2
            refs[s + 1][0][...] = pooled[:half]
            refs[s + 1][1][...] = pooled[half:]


def kernel(preds, target):
    preds = preds.astype(jnp.float32)
    target = target.astype(jnp.float32)
    B, C, D, H, W = preds.shape
    bc = B * C
    nb = _NB
    nblk = bc // nb

    x = preds.reshape(bc * D * H, W)
    y = target.reshape(bc * D * H, W)
    rows = nb * D * H
    out = pl.pallas_call(
        functools.partial(_loss_kernel, nb=nb, hw=H),
        out_shape=jax.ShapeDtypeStruct((nblk, 48, 128), jnp.float32),
        grid=(nblk,),
        in_specs=[
            pl.BlockSpec((rows, W), lambda i: (i, 0)),
            pl.BlockSpec((rows, W), lambda i: (i, 0)),
        ],
        out_specs=pl.BlockSpec((1, 48, 128), lambda i: (i, 0, 0)),
        scratch_shapes=[
            pltpu.VMEM((nb * D * (H >> (i + 1)), 128), jnp.float32)
            for i in range(_N_SCALES - 1) for _ in range(2)
        ] + [
            pltpu.VMEM((2 * nb * D * (H >> s), 128), jnp.float32)
            for s in range(_N_SCALES - 1)
        ],
        compiler_params=pltpu.CompilerParams(
            dimension_semantics=("parallel",),
            allow_input_fusion=[True, True]),
    )(x, y)

    abs_sum = jnp.sum(out[:, _N_SCALES * nb, 0])
    mcs_cols = []
    for s in range(_N_SCALES):
        hs = H >> s
        ho, wo = hs - _KW + 1, hs - _KW + 1
        wp = _wo_pad(wo)
        rows = out[:, s * nb:(s + 1) * nb, :]
        tot = jnp.sum(rows, axis=-1).reshape(bc)
        tot = tot - float(ho * (wp - wo))
        mcs_cols.append(tot * (1.0 / float(ho * wo)))
    mcs = jnp.stack(mcs_cols, axis=-1).reshape(B, C, _N_SCALES).mean(axis=1)
    mcs = jnp.maximum(mcs, 1e-6)
    betas = jnp.asarray(_BETAS, jnp.float32)[None, :]
    ms = jnp.mean(jnp.prod(mcs ** betas, axis=1))
    n = float(preds.size)
    loss = jnp.float32(0.0)
    loss = loss + 0.5 * (abs_sum / n)
    loss = loss + 0.5 * (1.0 - ms)
    return loss


# speedup vs baseline: 1.1889x; 1.1072x over previous
import functools

import jax
import jax.numpy as jnp
from jax.experimental import pallas as pl
from jax.experimental.pallas import tpu as pltpu

_K1 = 0.01
_K2 = 0.03
_DATA_RANGE = 1.0
_BETAS = (0.0448, 0.2856, 0.3001, 0.2363, 0.1333)
_N_SCALES = len(_BETAS)
_KW = 11

_NB = 8


def _wo_pad(wo):
    return 256 if wo > 128 else 128


def _band_right(w, wo, wp):
    j = jax.lax.broadcasted_iota(jnp.int32, (w, wp), 0)
    o = jax.lax.broadcasted_iota(jnp.int32, (w, wp), 1)
    return ((j >= o) & (j < o + _KW) & (o < wo)).astype(jnp.float32)


def _band_left(ho, h):
    o = jax.lax.broadcasted_iota(jnp.int32, (ho, h), 0)
    j = jax.lax.broadcasted_iota(jnp.int32, (ho, h), 1)
    return ((j >= o) & (j < o + _KW)).astype(jnp.float32)


def _pool_cols(w_store, w):
    j = jax.lax.broadcasted_iota(jnp.int32, (w_store, 128), 0)
    o = jax.lax.broadcasted_iota(jnp.int32, (w_store, 128), 1)
    sel = (j >= 2 * o) & (j < 2 * o + 2) & (o < w // 2) & (j < w)
    return jnp.where(sel, 0.25, 0.0).astype(jnp.float32)


def _loss_kernel(x_ref, y_ref, out_ref, *scratch, nb, hw):
    c1 = (_K1 * _DATA_RANGE) ** 2
    c2 = (_K2 * _DATA_RANGE) ** 2
    inv_n = 1.0 / float(3 * _KW * _KW)

    refs = [(x_ref, y_ref)] + [
        (scratch[2 * i], scratch[2 * i + 1]) for i in range(_N_SCALES - 1)
    ]
    tbufs = scratch[2 * (_N_SCALES - 1):]
    for s in range(_N_SCALES):
        hs = hw >> s
        ws = hw >> s
        stw = ws if s == 0 else 128
        xr, yr = refs[s]
        x = xr[...].reshape(nb, 3, hs, stw)
        y = yr[...].reshape(nb, 3, hs, stw)
        if s == 0:
            l1 = jnp.sum(jnp.abs(x - y))
            out_ref[0, _N_SCALES * nb:_N_SCALES * nb + 1, 0:1] = (
                l1.reshape(1, 1))
        ho, wo = hs - _KW + 1, ws - _KW + 1
        wp = _wo_pad(wo)

        x0, x1, x2 = x[:, 0], x[:, 1], x[:, 2]
        y0, y1, y2 = y[:, 0], y[:, 1], y[:, 2]
        zx = x0 + x1 + x2
        zy = y0 + y1 + y2
        zxx = x0 * x0 + x1 * x1 + x2 * x2
        zyy = y0 * y0 + y1 * y1 + y2 * y2
        zxy = x0 * y0 + x1 * y1 + x2 * y2

        aw = _band_right(stw, wo, wp)
        ah = _band_left(ho, hs)

        z2 = jnp.concatenate(
            [m.astype(jnp.bfloat16).reshape(nb * hs, stw)
             for m in (zx, zy, zxx, zyy, zxy)], axis=0
        )
        zw = jnp.dot(z2, aw.astype(jnp.bfloat16),
                     preferred_element_type=jnp.float32)
        zwb = zw.astype(jnp.bfloat16)
        ahb = ah.astype(jnp.bfloat16)

        for n in range(nb):
            wins = [
                jnp.dot(ahb, zwb[(m * nb + n) * hs:(m * nb + n + 1) * hs, :],
                        preferred_element_type=jnp.float32) * inv_n
                for m in range(5)
            ]
            mu_x, mu_y, exx, eyy, exy = wins
            sxx = exx - mu_x * mu_x
            syy = eyy - mu_y * mu_y
            sxy = exy - mu_x * mu_y
            cs_map = (2.0 * sxy + c2) / (sxx + syy + c2)
            if s == _N_SCALES - 1:
                cs_map = (2.0 * mu_x * mu_y + c1) / (
                    mu_x * mu_x + mu_y * mu_y + c1) * cs_map
            rv = jnp.sum(cs_map, axis=0, keepdims=True)
            if wp == 256:
                rv = rv[:, :128] + rv[:, 128:]
            out_ref[0, s * nb + n:s * nb + n + 1, :] = rv

        if s < _N_SCALES - 1:
            h2 = hs // 2
            pw = _pool_cols(stw, ws)
            cat = jnp.concatenate(
                [x.astype(jnp.bfloat16).reshape(nb * 3 * hs, stw),
                 y.astype(jnp.bfloat16).reshape(nb * 3 * hs, stw)],
                axis=0)
            tb = tbufs[s]
            tb[...] = jnp.dot(cat, pw.astype(jnp.bfloat16),
                              preferred_element_type=jnp.float32)
            pooled = tb[0::2, :] + tb[1::2, :]
            half = nb * 3 * h2
            refs[s + 1][0][...] = pooled[:half]
            refs[s + 1][1][...] = pooled[half:]


def kernel(preds, target):
    preds = preds.astype(jnp.float32)
    target = target.astype(jnp.float32)
    B, C, D, H, W = preds.shape
    bc = B * C
    nb = _NB
    nblk = bc // nb

    x = preds.reshape(bc * D * H, W)
    y = target.reshape(bc * D * H, W)
    rows = nb * D * H
    out = pl.pallas_call(
        functools.partial(_loss_kernel, nb=nb, hw=H),
        out_shape=jax.ShapeDtypeStruct((nblk, 48, 128), jnp.float32),
        grid=(nblk,),
        in_specs=[
            pl.BlockSpec((rows, W), lambda i: (i, 0)),
            pl.BlockSpec((rows, W), lambda i: (i, 0)),
        ],
        out_specs=pl.BlockSpec((1, 48, 128), lambda i: (i, 0, 0)),
        scratch_shapes=[
            pltpu.VMEM((nb * D * (H >> (i + 1)), 128), jnp.float32)
            for i in range(_N_SCALES - 1) for _ in range(2)
        ] + [
            pltpu.VMEM((2 * nb * D * (H >> s), 128), jnp.float32)
            for s in range(_N_SCALES - 1)
        ],
        compiler_params=pltpu.CompilerParams(
            dimension_semantics=("parallel",),
            allow_input_fusion=[True, True]),
    )(x, y)

    abs_sum = jnp.sum(out[:, _N_SCALES * nb, 0])
    mcs_cols = []
    for s in range(_N_SCALES):
        hs = H >> s
        ho, wo = hs - _KW + 1, hs - _KW + 1
        wp = _wo_pad(wo)
        rows = out[:, s * nb:(s + 1) * nb, :]
        tot = jnp.sum(rows, axis=-1).reshape(bc)
        tot = tot - float(ho * (wp - wo))
        mcs_cols.append(tot * (1.0 / float(ho * wo)))
    mcs = jnp.stack(mcs_cols, axis=-1).reshape(B, C, _N_SCALES).mean(axis=1)
    mcs = jnp.maximum(mcs, 1e-6)
    betas = jnp.asarray(_BETAS, jnp.float32)[None, :]
    ms = jnp.mean(jnp.prod(mcs ** betas, axis=1))
    n = float(preds.size)
    loss = jnp.float32(0.0)
    loss = loss + 0.5 * (abs_sum / n)
    loss = loss + 0.5 * (1.0 - ms)
    return loss
